# Initial kernel scaffold; baseline (speedup 1.0000x reference)
import functools

import jax
import jax.numpy as jnp
from jax import lax
from jax.experimental import pallas as pl
from jax.experimental.pallas import tpu as pltpu
from jax.experimental.pallas import tpu_sc as plsc

_N = 10000
_E = 320000
_D = 128

_NC = 2
_NS = 16
_NW = _NC * _NS
_EPW = _E // _NW
_CH = 80
_NCH = _EPW // _CH
_RPT = _N // _NS
_CW = 16

_BLK = 1000


def _norm_body(h_ref, o_ref):
    h = h_ref[...]
    n = jnp.sqrt(jnp.sum(h * h, axis=1, keepdims=True)) + 1e-8
    o_ref[...] = h / n


def _normalize(H):
    return pl.pallas_call(
        _norm_body,
        grid=(_N // _BLK,),
        in_specs=[pl.BlockSpec((_BLK, _D), lambda i: (i, 0))],
        out_specs=pl.BlockSpec((_BLK, _D), lambda i: (i, 0)),
        out_shape=jax.ShapeDtypeStruct((_N, _D), jnp.float32),
    )(H)


_mesh = plsc.VectorSubcoreMesh(core_axis_name="c", subcore_axis_name="s")


@functools.partial(
    pl.kernel,
    out_type=(
        jax.ShapeDtypeStruct((_NC, _N, _D), jnp.float32),
        jax.ShapeDtypeStruct((_NC, _N, _CW), jnp.float32),
    ),
    mesh=_mesh,
    scratch_types=[
        pltpu.VMEM((_NCH, _CH), jnp.int32),
        pltpu.VMEM((_NCH, _CH), jnp.int32),
        pltpu.VMEM((_CH, _D), jnp.float32),
        pltpu.VMEM((_CH, _CW), jnp.float32),
        pltpu.VMEM((125, _D), jnp.float32),
        pltpu.VMEM((_RPT, _CW), jnp.float32),
        pltpu.VMEM_SHARED((_N, _D), jnp.float32),
        pltpu.VMEM_SHARED((_N, _CW), jnp.float32),
        pltpu.SemaphoreType.DMA,
    ],
)
def _segsum(hn, src_r, dst_r, zsum, zcnt, sum_out, cnt_out,
            src_v, dst_v, rows_v, ones_v, stage_v, cstage_v, acc_sh, cnt_sh,
            sem):
    cid = lax.axis_index("c")
    sid = lax.axis_index("s")
    wid = cid * _NS + sid

    for r in range(_CH):
        ones_v[r, :] = jnp.ones((_CW,), jnp.float32)

    pltpu.sync_copy(zsum.at[pl.ds(sid * _RPT, _RPT)],
                    acc_sh.at[pl.ds(sid * _RPT, _RPT)])
    pltpu.sync_copy(zcnt.at[pl.ds(sid * _RPT, _RPT)],
                    cnt_sh.at[pl.ds(sid * _RPT, _RPT)])

    pltpu.sync_copy(src_r.at[wid], src_v)
    pltpu.sync_copy(dst_r.at[wid], dst_v)
    plsc.subcore_barrier()

    def step(j, carry):
        pltpu.async_copy(hn.at[dst_v.at[j]], rows_v, sem).wait()
        pltpu.sync_copy(rows_v, acc_sh.at[src_v.at[j]], add=True)
        pltpu.sync_copy(ones_v, cnt_sh.at[src_v.at[j]], add=True)
        return carry

    lax.fori_loop(0, _NCH, step, 0)
    plsc.subcore_barrier()

    for b in range(_RPT // 125):
        r0 = sid * _RPT + b * 125
        pltpu.sync_copy(acc_sh.at[pl.ds(r0, 125)], stage_v)
        pltpu.sync_copy(stage_v, sum_out.at[cid, pl.ds(r0, 125)])
    c0 = sid * _RPT
    pltpu.sync_copy(cnt_sh.at[pl.ds(c0, _RPT)], cstage_v)
    pltpu.sync_copy(cstage_v, cnt_out.at[cid, pl.ds(c0, _RPT)])


def _fin_body(a_ref, hn_ref, s_ref, c_ref, o_ref):
    hn = hn_ref[...]
    s = s_ref[0] + s_ref[1]
    cnt = c_ref[0, :, 0:1] + c_ref[1, :, 0:1]
    curv = jnp.clip(a_ref[0], 1e-8, 1.0)
    cos = jnp.sum(hn * s, axis=1, keepdims=True)
    v = (s - cos * hn) * (curv / jnp.maximum(cnt, 1.0))
    n2 = jnp.sqrt(jnp.sum(v * v, axis=1, keepdims=True)) + 1e-8
    o_ref[...] = v / n2


def _finalize(a, hn, sums, cnts):
    return pl.pallas_call(
        _fin_body,
        grid=(_N // _BLK,),
        in_specs=[
            pl.BlockSpec(memory_space=pltpu.SMEM),
            pl.BlockSpec((_BLK, _D), lambda i: (i, 0)),
            pl.BlockSpec((_NC, _BLK, _D), lambda i: (0, i, 0)),
            pl.BlockSpec((_NC, _BLK, _CW), lambda i: (0, i, 0)),
        ],
        out_specs=pl.BlockSpec((_BLK, _D), lambda i: (i, 0)),
        out_shape=jax.ShapeDtypeStruct((_N, _D), jnp.float32),
    )(a, hn, sums, cnts)


@jax.jit
def kernel(t, H, edge_index, a):
    src = edge_index[0].astype(jnp.int32).reshape(_NW, _NCH, _CH)
    dst = edge_index[1].astype(jnp.int32).reshape(_NW, _NCH, _CH)
    hn = _normalize(H)
    zsum = jnp.zeros((_N, _D), jnp.float32)
    zcnt = jnp.zeros((_N, _CW), jnp.float32)
    sums, cnts = _segsum(hn, src, dst, zsum, zcnt)
    return _finalize(jnp.reshape(a, (1,)), hn, sums, cnts)


# speedup vs baseline: 13.4600x; 13.4600x over previous
import functools

import jax
import jax.numpy as jnp
from jax import lax
from jax.experimental import pallas as pl
from jax.experimental.pallas import tpu as pltpu
from jax.experimental.pallas import tpu_sc as plsc

_N = 10000
_E = 320000
_D = 128

_NC = 2
_NS = 16
_NW = _NC * _NS
_EPW = _E // _NW
_CH = 100
_NCH = _EPW // _CH
_NP = 10240
_RPT = _NP // _NS
_CW = 16
_DW = _D + _CW

_BLK = 1000
_ZBLK = _NP // 10


def _norm_body(h_ref, o_ref, z_ref):
    h = h_ref[...]
    n = jnp.sqrt(jnp.sum(h * h, axis=1, keepdims=True)) + 1e-8
    o_ref[:, :_D] = h / n
    o_ref[:, _D:] = jnp.ones((_BLK, _CW), jnp.float32)
    z_ref[...] = jnp.zeros((_ZBLK, _DW), jnp.float32)


def _normalize(H):
    return pl.pallas_call(
        _norm_body,
        grid=(_N // _BLK,),
        in_specs=[pl.BlockSpec((_BLK, _D), lambda i: (i, 0))],
        out_specs=[
            pl.BlockSpec((_BLK, _DW), lambda i: (i, 0)),
            pl.BlockSpec((_ZBLK, _DW), lambda i: (i, 0)),
        ],
        out_shape=[
            jax.ShapeDtypeStruct((_N, _DW), jnp.float32),
            jax.ShapeDtypeStruct((_NP, _DW), jnp.float32),
        ],
    )(H)


@functools.cache
def _build_segsum():
    mesh = plsc.VectorSubcoreMesh(core_axis_name="c", subcore_axis_name="s",
                                  num_cores=_NC, num_subcores=_NS)

    @functools.partial(
        pl.kernel,
        out_type=jax.ShapeDtypeStruct((_NC, _NP, _DW), jnp.float32),
        mesh=mesh,
        compiler_params=pltpu.CompilerParams(use_tc_tiling_on_sc=False),
        scratch_types=[
            pltpu.VMEM((2, _CH), jnp.int32),
            pltpu.VMEM((2, _CH), jnp.int32),
            pltpu.VMEM((_CH, _DW), jnp.float32),
            pltpu.VMEM((_CH, _DW), jnp.float32),
            pltpu.VMEM_SHARED((_NP, _DW), jnp.float32),
            pltpu.SemaphoreType.DMA,
            pltpu.SemaphoreType.DMA,
        ],
    )
    def _segsum(hn, idx_r, zero, sum_out,
                idx0_v, idx1_v, rows0_v, rows1_v, acc_sh, sem0, sem1):
        cid = lax.axis_index("c")
        sid = lax.axis_index("s")
        wid = cid * _NS + sid

        z0 = pl.multiple_of(sid * _RPT, 8)
        pltpu.sync_copy(zero.at[pl.ds(z0, _RPT)], acc_sh.at[pl.ds(z0, _RPT)])

        pltpu.sync_copy(idx_r.at[wid, 0], idx0_v)
        plsc.subcore_barrier()
        pltpu.async_copy(hn.at[idx0_v.at[1]], rows0_v, sem0)

        def step(j2, carry):
            j = 2 * j2
            pltpu.sync_copy(idx_r.at[wid, j + 1], idx1_v)
            pltpu.async_copy(hn.at[idx1_v.at[1]], rows1_v, sem1)
            pltpu.make_async_copy(hn.at[idx0_v.at[1]], rows0_v, sem0).wait()
            pltpu.sync_copy(rows0_v, acc_sh.at[idx0_v.at[0]], add=True)

            @pl.when(j2 < _NCH // 2 - 1)
            def _():
                pltpu.sync_copy(idx_r.at[wid, j + 2], idx0_v)
                pltpu.async_copy(hn.at[idx0_v.at[1]], rows0_v, sem0)

            pltpu.make_async_copy(hn.at[idx1_v.at[1]], rows1_v, sem1).wait()
            pltpu.sync_copy(rows1_v, acc_sh.at[idx1_v.at[0]], add=True)
            return carry

        lax.fori_loop(0, _NCH // 2, step, 0)
        plsc.subcore_barrier()

        for b in range(_RPT // 80):
            r0 = pl.multiple_of(sid * _RPT + b * 80, 8)
            pltpu.sync_copy(acc_sh.at[pl.ds(r0, 80)], rows0_v.at[pl.ds(0, 80)])
            pltpu.sync_copy(rows0_v.at[pl.ds(0, 80)],
                            sum_out.at[cid, pl.ds(r0, 80)])

    return _segsum


def _fin_body(a_ref, hn_ref, s_ref, o_ref):
    hn = hn_ref[:, :_D]
    s = s_ref[0, :, :_D] + s_ref[1, :, :_D]
    cnt = s_ref[0, :, _D:_D + 1] + s_ref[1, :, _D:_D + 1]
    curv = jnp.clip(a_ref[0], 1e-8, 1.0)
    cos = jnp.sum(hn * s, axis=1, keepdims=True)
    v = (s - cos * hn) * (curv / jnp.maximum(cnt, 1.0))
    n2 = jnp.sqrt(jnp.sum(v * v, axis=1, keepdims=True)) + 1e-8
    o_ref[...] = v / n2


def _finalize(a, hn, sums):
    return pl.pallas_call(
        _fin_body,
        grid=(_N // _BLK,),
        in_specs=[
            pl.BlockSpec(memory_space=pltpu.SMEM),
            pl.BlockSpec((_BLK, _DW), lambda i: (i, 0)),
            pl.BlockSpec((_NC, _BLK, _DW), lambda i: (0, i, 0)),
        ],
        out_specs=pl.BlockSpec((_BLK, _D), lambda i: (i, 0)),
        out_shape=jax.ShapeDtypeStruct((_N, _D), jnp.float32),
    )(a, hn, sums)


@jax.jit
def kernel(t, H, edge_index, a):
    idx = edge_index.astype(jnp.int32).reshape(2, _NW, _NCH, _CH)
    idx = jnp.transpose(idx, (1, 2, 0, 3))
    hn, zero = _normalize(H)
    sums = _build_segsum()(hn, idx, zero)
    return _finalize(jnp.reshape(a, (1,)), hn, sums)
